# Initial kernel scaffold; baseline (speedup 1.0000x reference)
import jax
import jax.numpy as jnp
from jax.experimental import pallas as pl


def kernel(x, edge_index, edge_type, node_to_graph, W_init, W_msg, b_msg, W_upd, b_upd, W_score, W_val, W_headout, W_mean):
    raise NotImplementedError("write your pallas kernel here")


# speedup vs baseline: 3.5405x; 3.5405x over previous
import functools

import jax
import jax.numpy as jnp
from jax import lax
from jax.experimental import pallas as pl
from jax.experimental.pallas import tpu as pltpu
from jax.experimental.pallas import tpu_sc as plsc

N = 10000
E = 160000
ATOM = 128
H = 128
L = 10
T = 3
G = 200
HEADS = 12
HD = 64
OUT = 512
DALL = (L + 1) * H

NW = 32
NT = 320
NTH = 160
NPAD = NW * NT
DUMP = NTH
ACCR = NTH + 1
CHUNK = 128
NEG = -1e9

BLK = 2048
GRID = NPAD // BLK
GP = 256

f32 = jnp.float32
i32 = jnp.int32


def _full(shape):
    nd = len(shape)
    return pl.BlockSpec(shape, lambda i: (0,) * nd)


def _rows(shape):
    return pl.BlockSpec(shape, lambda i: (i,) + (0,) * (len(shape) - 1))


def _init_body(x_ref, wi_ref, ws_ref, wd_ref, bm_ref, h_ref, a_ref, b_ref):
    h = jnp.dot(x_ref[...], wi_ref[...], preferred_element_type=f32)
    h_ref[...] = h
    a_ref[...] = jnp.dot(h, ws_ref[...], preferred_element_type=f32) + bm_ref[...]
    b_ref[...] = jnp.dot(h, wd_ref[...], preferred_element_type=f32)


def _tc_init(xp, W_init, Ws, Wd, bm):
    return pl.pallas_call(
        _init_body,
        grid=(GRID,),
        in_specs=[_rows((BLK, ATOM)), _full((ATOM, H)), _full((H, T * H)),
                  _full((H, T * H)), _full((1, T * H))],
        out_specs=[_rows((BLK, H)), _rows((BLK, T * H)), _rows((BLK, T * H))],
        out_shape=[jax.ShapeDtypeStruct((NPAD, H), f32),
                   jax.ShapeDtypeStruct((NPAD, T * H), f32),
                   jax.ShapeDtypeStruct((NPAD, T * H), f32)],
    )(xp, W_init, Ws, Wd, bm)


def _scal_body(deg_ref, scal_ref):
    deg = deg_ref[...]
    ld = jnp.log(1.0 + deg)
    rid = lax.broadcasted_iota(i32, (NPAD, 1), 0)
    valid = rid < N
    delta = jnp.sum(jnp.where(valid, ld, 0.0)) / N
    safe = jnp.where(ld > 0, ld, 1.0)
    inv = 1.0 / jnp.maximum(deg, 1.0)
    amp = ld / delta
    att = delta / safe
    pos = (deg > 0).astype(f32)
    scal_ref[...] = jnp.concatenate([inv, amp, att, pos], axis=1)


def _tc_scal(deg):
    return pl.pallas_call(
        _scal_body,
        grid=(1,),
        in_specs=[_full((NPAD, 1))],
        out_specs=_full((NPAD, 4)),
        out_shape=jax.ShapeDtypeStruct((NPAD, 4), f32),
    )(deg)


def _upd_core(h_ref, s_ref, m_ref, sc_ref, w3_ref, bu_ref):
    s = s_ref[...]
    sc = sc_ref[...]
    inv, amp, att, pos = sc[:, 0:1], sc[:, 1:2], sc[:, 2:3], sc[:, 3:4]
    base = jnp.concatenate([s * inv, m_ref[...] * pos, s], axis=1)
    u = jnp.dot(base, w3_ref[...], preferred_element_type=f32)
    pre = u[:, 0:H] + amp * u[:, H:2 * H] + att * u[:, 2 * H:3 * H] + bu_ref[...]
    return jnp.maximum(pre, 0.0) + h_ref[...]


def _upd_body(h_ref, s_ref, m_ref, sc_ref, w3_ref, bu_ref, ws_ref, wd_ref,
              bm_ref, hn_ref, a_ref, b_ref):
    hn = _upd_core(h_ref, s_ref, m_ref, sc_ref, w3_ref, bu_ref)
    hn_ref[...] = hn
    a_ref[...] = jnp.dot(hn, ws_ref[...], preferred_element_type=f32) + bm_ref[...]
    b_ref[...] = jnp.dot(hn, wd_ref[...], preferred_element_type=f32)


def _upd_last_body(h_ref, s_ref, m_ref, sc_ref, w3_ref, bu_ref, hn_ref):
    hn_ref[...] = _upd_core(h_ref, s_ref, m_ref, sc_ref, w3_ref, bu_ref)


def _tc_upd(h, s_agg, m_agg, scal, W3, bu, Ws, Wd, bm):
    return pl.pallas_call(
        _upd_body,
        grid=(GRID,),
        in_specs=[_rows((BLK, H)), _rows((BLK, H)), _rows((BLK, H)),
                  _rows((BLK, 4)), _full((3 * H, 3 * H)), _full((1, H)),
                  _full((H, T * H)), _full((H, T * H)), _full((1, T * H))],
        out_specs=[_rows((BLK, H)), _rows((BLK, T * H)), _rows((BLK, T * H))],
        out_shape=[jax.ShapeDtypeStruct((NPAD, H), f32),
                   jax.ShapeDtypeStruct((NPAD, T * H), f32),
                   jax.ShapeDtypeStruct((NPAD, T * H), f32)],
    )(h, s_agg, m_agg, scal, W3, bu, Ws, Wd, bm)


def _tc_upd_last(h, s_agg, m_agg, scal, W3, bu):
    return pl.pallas_call(
        _upd_last_body,
        grid=(GRID,),
        in_specs=[_rows((BLK, H)), _rows((BLK, H)), _rows((BLK, H)),
                  _rows((BLK, 4)), _full((3 * H, 3 * H)), _full((1, H))],
        out_specs=_rows((BLK, H)),
        out_shape=jax.ShapeDtypeStruct((NPAD, H), f32),
    )(h, s_agg, m_agg, scal, W3, bu)


def _r1_body(ha_ref, ng_ref, wsc_ref, wv_ref, wsum_ref, hsum_ref, cnt_ref):
    step = pl.program_id(0)
    ha = ha_ref[...]
    scores = jax.nn.sigmoid(jnp.dot(ha, wsc_ref[...], preferred_element_type=f32))
    vals = jnp.dot(ha, wv_ref[...], preferred_element_type=f32)
    sv = jnp.concatenate(
        [scores[:, k:k + 1] * vals[:, k * HD:(k + 1) * HD] for k in range(HEADS)],
        axis=1)
    gi = lax.broadcasted_iota(i32, (BLK, GP), 1)
    oh = (gi == ng_ref[...]).astype(f32)
    dn = (((0,), (0,)), ((), ()))
    w = lax.dot_general(oh, sv, dn, preferred_element_type=f32)
    hs = lax.dot_general(oh, ha, dn, preferred_element_type=f32)
    c = lax.dot_general(oh, jnp.ones((BLK, 8), f32), dn, preferred_element_type=f32)

    @pl.when(step == 0)
    def _():
        wsum_ref[...] = w
        hsum_ref[...] = hs
        cnt_ref[...] = c

    @pl.when(step > 0)
    def _():
        wsum_ref[...] += w
        hsum_ref[...] += hs
        cnt_ref[...] += c


def _tc_r1(h_all, ngp, W_score, W_val):
    return pl.pallas_call(
        _r1_body,
        grid=(GRID,),
        in_specs=[_rows((BLK, DALL)), _rows((BLK, 1)),
                  _full((DALL, HEADS)), _full((DALL, HEADS * HD))],
        out_specs=[_full((GP, HEADS * HD)), _full((GP, DALL)), _full((GP, 8))],
        out_shape=[jax.ShapeDtypeStruct((GP, HEADS * HD), f32),
                   jax.ShapeDtypeStruct((GP, DALL), f32),
                   jax.ShapeDtypeStruct((GP, 8), f32)],
    )(h_all, ngp, W_score, W_val)


def _r2_body(w_ref, hs_ref, c_ref, who_ref, wm_ref, o_ref):
    c = jnp.maximum(c_ref[:, 0:1], 1.0)
    ms = hs_ref[...] / c
    o_ref[...] = (jnp.dot(w_ref[...], who_ref[...], preferred_element_type=f32)
                  + jnp.dot(ms, wm_ref[...], preferred_element_type=f32))


def _tc_r2(wsum, hsum, cnt, W_headout, W_mean):
    return pl.pallas_call(
        _r2_body,
        grid=(1,),
        in_specs=[_full((GP, HEADS * HD)), _full((GP, DALL)), _full((GP, 8)),
                  _full((HEADS * HD, OUT)), _full((DALL, OUT))],
        out_specs=_full((GP, OUT)),
        out_shape=jax.ShapeDtypeStruct((GP, OUT), f32),
    )(wsum, hsum, cnt, W_headout, W_mean)


def _sc_edge_call(A2, B2, gA, gB, dstv, est, with_deg):
    mesh = plsc.VectorSubcoreMesh(core_axis_name="c", subcore_axis_name="s",
                                  num_cores=2, num_subcores=16)
    out_type = [jax.ShapeDtypeStruct((NPAD, H), f32),
                jax.ShapeDtypeStruct((NPAD, H), f32)]
    scratch = [
        pltpu.VMEM((ACCR, H), f32),
        pltpu.VMEM((ACCR, H), f32),
        pltpu.VMEM((CHUNK,), i32),
        pltpu.VMEM((CHUNK,), i32),
        pltpu.VMEM((CHUNK,), i32),
        pltpu.VMEM((CHUNK,), i32),
        pltpu.VMEM((CHUNK, H), f32),
        pltpu.VMEM((CHUNK, H), f32),
        pltpu.VMEM((NW, 16), i32),
        pltpu.SemaphoreType.DMA,
        pltpu.SemaphoreType.DMA,
    ]
    if with_deg:
        out_type.append(jax.ShapeDtypeStruct((NPAD, 16), f32))
        scratch.append(pltpu.VMEM((ACCR, 16), f32))

    def body(a_hbm, b_hbm, ga_hbm, gb_hbm, dst_hbm, est_hbm, *rest):
        if with_deg:
            s_out, m_out, deg_out = rest[0], rest[1], rest[2]
            (acc_s, acc_m, idxa, idxb, dbuf, dloc, bufa, bufb, estv,
             sema, semb, dega) = rest[3:]
        else:
            s_out, m_out = rest[0], rest[1]
            (acc_s, acc_m, idxa, idxb, dbuf, dloc, bufa, bufb, estv,
             sema, semb) = rest[2:]
        wid = lax.axis_index("s") * 2 + lax.axis_index("c")
        pltpu.sync_copy(est_hbm, estv)
        v = estv[wid, pl.ds(0, 16)]
        one0 = (lax.broadcasted_iota(i32, (16,), 0) == 0).astype(f32)

        for half in range(2):
            nlo = wid * NT + half * NTH
            lo = v[2 * half]
            hi = v[2 * half + 1]

            def zero_body(r, _):
                for j in range(H // 16):
                    sl = pl.ds(j * 16, 16)
                    acc_s[r, sl] = jnp.zeros((16,), f32)
                    acc_m[r, sl] = jnp.full((16,), NEG, f32)
                return 0
            lax.fori_loop(0, ACCR, zero_body, 0)
            if with_deg:
                def zdeg(r, _):
                    dega[r, pl.ds(0, 16)] = jnp.zeros((16,), f32)
                    return 0
                lax.fori_loop(0, ACCR, zdeg, 0)

            c0 = lo // CHUNK
            c1 = (hi + CHUNK - 1) // CHUNK

            def chunk_body(c, _):
                e0 = c * CHUNK
                pltpu.sync_copy(ga_hbm.at[pl.ds(e0, CHUNK)], idxa)
                pltpu.sync_copy(gb_hbm.at[pl.ds(e0, CHUNK)], idxb)
                pltpu.sync_copy(dst_hbm.at[pl.ds(e0, CHUNK)], dbuf)
                cpa = pltpu.async_copy(a_hbm.at[idxa], bufa, sema)
                cpb = pltpu.async_copy(b_hbm.at[idxb], bufb, semb)
                for vv in range(CHUNK // 16):
                    sl = pl.ds(vv * 16, 16)
                    d16 = dbuf[sl]
                    ok = (d16 >= nlo) & (d16 < nlo + NTH)
                    dloc[sl] = jnp.where(ok, d16 - nlo, DUMP)
                cpa.wait()
                cpb.wait()

                def grp_body(g, _):
                    dl16 = dloc[pl.ds(g * 16, 16)]
                    for lane in range(16):
                        dl = dl16[lane]
                        row = g * 16 + lane
                        if with_deg:
                            sl0 = pl.ds(0, 16)
                            dega[dl, sl0] = dega[dl, sl0] + one0
                        for j in range(H // 16):
                            sl = pl.ds(j * 16, 16)
                            m = jnp.maximum(bufa[row, sl] + bufb[row, sl], 0.0)
                            acc_s[dl, sl] = acc_s[dl, sl] + m
                            acc_m[dl, sl] = jnp.maximum(acc_m[dl, sl], m)
                    return 0
                lax.fori_loop(0, CHUNK // 16, grp_body, 0)
                return 0
            lax.fori_loop(c0, c1, chunk_body, 0)

            pltpu.sync_copy(acc_s.at[pl.ds(0, NTH)], s_out.at[pl.ds(nlo, NTH)])
            pltpu.sync_copy(acc_m.at[pl.ds(0, NTH)], m_out.at[pl.ds(nlo, NTH)])
            if with_deg:
                pltpu.sync_copy(dega.at[pl.ds(0, NTH)],
                                deg_out.at[pl.ds(nlo, NTH)])

    k = pl.kernel(body, out_type=out_type, mesh=mesh, scratch_types=scratch,
                  compiler_params=pltpu.CompilerParams(needs_layout_passes=False))
    return k(A2, B2, gA, gB, dstv, est)


def kernel(x, edge_index, edge_type, node_to_graph, W_init, W_msg, b_msg,
           W_upd, b_upd, W_score, W_val, W_headout, W_mean):
    src = edge_index[0]
    dst = edge_index[1]
    perm = jnp.argsort(dst)
    srcs = src[perm].astype(i32)
    dsts = dst[perm].astype(i32)
    ets = edge_type[perm].astype(i32)
    gA = srcs * T + ets
    gB = dsts * T + ets
    bounds = jnp.arange(2 * NW + 1, dtype=i32) * NTH
    e65 = jnp.searchsorted(dsts, bounds, side="left").astype(i32)
    est = jnp.stack([e65[0:2 * NW:2], e65[1:2 * NW:2], e65[1:2 * NW:2],
                     e65[2:2 * NW + 1:2]] + [jnp.zeros((NW,), i32)] * 12,
                    axis=1)
    xp = jnp.pad(x, ((0, NPAD - N), (0, 0)))

    Ws_all = jnp.transpose(W_msg[:, :, :H, :], (0, 2, 1, 3)).reshape(L, H, T * H)
    Wd_all = jnp.transpose(W_msg[:, :, H:, :], (0, 2, 1, 3)).reshape(L, H, T * H)
    bm_all = b_msg.reshape(L, 1, T * H)
    W3_all = jnp.concatenate(
        [W_upd[:, 0:3 * H], W_upd[:, 3 * H:6 * H], W_upd[:, 6 * H:9 * H]], axis=2)
    bu_all = b_upd.reshape(L, 1, H)

    h, A, B = _tc_init(xp, W_init, Ws_all[0], Wd_all[0], bm_all[0])
    states = [h]
    scal = None
    for l in range(L):
        A2 = A.reshape(NPAD * T, H)
        B2 = B.reshape(NPAD * T, H)
        if l == 0:
            s_agg, m_agg, deg = _sc_edge_call(A2, B2, gA, gB, dsts, est, True)
            scal = _tc_scal(deg[:, 0:1])
        else:
            s_agg, m_agg = _sc_edge_call(A2, B2, gA, gB, dsts, est, False)
        if l < L - 1:
            h, A, B = _tc_upd(h, s_agg, m_agg, scal, W3_all[l], bu_all[l],
                              Ws_all[l + 1], Wd_all[l + 1], bm_all[l + 1])
        else:
            h = _tc_upd_last(h, s_agg, m_agg, scal, W3_all[l], bu_all[l])
        states.append(h)

    h_all = jnp.concatenate(states, axis=1)
    ngp = jnp.concatenate(
        [node_to_graph.astype(i32), jnp.full((NPAD - N,), GP - 1, i32)]
    ).reshape(NPAD, 1)
    wsum, hsum, cnt = _tc_r1(h_all, ngp, W_score, W_val)
    o = _tc_r2(wsum, hsum, cnt, W_headout, W_mean)
    return o[:G]
